# Initial kernel scaffold; baseline (speedup 1.0000x reference)
import jax
import jax.numpy as jnp
from jax.experimental import pallas as pl


def kernel(atoms, deg_slice, membership, deg_adj_1, deg_adj_2, deg_adj_3, deg_adj_4, deg_adj_5, deg_adj_6, deg_adj_7, deg_adj_8, deg_adj_9, deg_adj_10):
    raise NotImplementedError("write your pallas kernel here")


# speedup vs baseline: 1.6977x; 1.6977x over previous
import jax
import jax.numpy as jnp
from jax import lax
from jax.experimental import pallas as pl
from jax.experimental.pallas import tpu as pltpu
from jax.experimental.pallas import tpu_sc as plsc

N = 100000
D = 128
MAX_DEG = 10
C0 = 10000
CD = 9000

NC = 2
NS = 16
NW = NC * NS

G = 24
NCHUNK = CD // G
TRIPS = (NCHUNK + NW - 1) // NW

G0 = 80
NCHUNK0 = C0 // G0
TRIPS0 = (NCHUNK0 + NW - 1) // NW

_mesh = plsc.VectorSubcoreMesh(
    core_axis_name="c", subcore_axis_name="s", num_cores=NC, num_subcores=NS
)


def _body(atoms, adj1, adj2, adj3, adj4, adj5, adj6, adj7, adj8, adj9, adj10,
          out, selfb, gb, ob, ib, cb, sem):
    adjs = [adj1, adj2, adj3, adj4, adj5, adj6, adj7, adj8, adj9, adj10]
    wid = lax.axis_index("s") * NC + lax.axis_index("c")

    def copy_body(t, carry):
        chunk = wid + NW * t

        @pl.when(chunk < NCHUNK0)
        def _():
            base = chunk * G0
            pltpu.sync_copy(atoms.at[pl.ds(base, G0)], cb)
            pltpu.sync_copy(cb, out.at[pl.ds(base, G0)])

        return carry

    lax.fori_loop(0, TRIPS0, copy_body, 0)

    for d in range(1, MAX_DEG + 1):
        adj = adjs[d - 1]
        row0 = C0 + (d - 1) * CD
        gsl = gb.at[pl.ds(0, G * d)]
        isl = ib.at[pl.ds(0, G * d)]

        def chunk_body(t, carry, adj=adj, row0=row0, gsl=gsl, isl=isl, d=d):
            chunk = wid + NW * t

            @pl.when(chunk < NCHUNK)
            def _():
                rbase = row0 + chunk * G
                pltpu.sync_copy(atoms.at[pl.ds(rbase, G)], selfb)
                pltpu.sync_copy(adj.at[pl.ds(chunk * G * d, G * d)], isl)
                pltpu.async_copy(atoms.at[isl], gsl, sem).wait()

                def row_body(i, c2):
                    for c in range(D // 16):
                        sl = pl.ds(c * 16, 16)
                        v = selfb[i, sl]
                        for j in range(d):
                            v = jnp.maximum(v, gb[i * d + j, sl])
                        ob[i, sl] = v
                    return c2

                lax.fori_loop(0, G, row_body, 0)
                pltpu.sync_copy(ob, out.at[pl.ds(rbase, G)])

            return carry

        lax.fori_loop(0, TRIPS, chunk_body, 0)


_pool = pl.kernel(
    _body,
    out_type=jax.ShapeDtypeStruct((N, D), jnp.float32),
    mesh=_mesh,
    scratch_types=[
        pltpu.VMEM((G, D), jnp.float32),
        pltpu.VMEM((G * MAX_DEG, D), jnp.float32),
        pltpu.VMEM((G, D), jnp.float32),
        pltpu.VMEM((G * MAX_DEG,), jnp.int32),
        pltpu.VMEM((G0, D), jnp.float32),
        pltpu.SemaphoreType.DMA,
    ],
)


def kernel(atoms, deg_slice, membership, deg_adj_1, deg_adj_2, deg_adj_3,
           deg_adj_4, deg_adj_5, deg_adj_6, deg_adj_7, deg_adj_8,
           deg_adj_9, deg_adj_10):
    flats = [a.reshape(-1) for a in
             (deg_adj_1, deg_adj_2, deg_adj_3, deg_adj_4, deg_adj_5,
              deg_adj_6, deg_adj_7, deg_adj_8, deg_adj_9, deg_adj_10)]
    return _pool(atoms, *flats)


# speedup vs baseline: 2.9856x; 1.7586x over previous
import jax
import jax.numpy as jnp
from jax import lax
from jax.experimental import pallas as pl
from jax.experimental.pallas import tpu as pltpu
from jax.experimental.pallas import tpu_sc as plsc

N = 100000
D = 128
MAX_DEG = 10
C0 = 10000
CD = 9000

NC = 2
NS = 16
NW = NC * NS

G = 24
NCHUNK = CD // G
TRIPS = (NCHUNK + NW - 1) // NW

G0 = 80
NCHUNK0 = C0 // G0
TRIPS0 = (NCHUNK0 + NW - 1) // NW

_mesh = plsc.VectorSubcoreMesh(
    core_axis_name="c", subcore_axis_name="s", num_cores=NC, num_subcores=NS
)


def _body(atoms, adj1, adj2, adj3, adj4, adj5, adj6, adj7, adj8, adj9, adj10,
          out, sb0, sb1, gb0, gb1, ob0, ob1, ib0, ib1, cb,
          gsem0, gsem1, ssem0, ssem1, osem0, osem1):
    adjs = [adj1, adj2, adj3, adj4, adj5, adj6, adj7, adj8, adj9, adj10]
    wid = lax.axis_index("s") * NC + lax.axis_index("c")

    def copy_body(t, carry):
        chunk = wid + NW * t

        @pl.when(chunk < NCHUNK0)
        def _():
            base = chunk * G0
            pltpu.sync_copy(atoms.at[pl.ds(base, G0)], cb)
            pltpu.sync_copy(cb, out.at[pl.ds(base, G0)])

        return carry

    lax.fori_loop(0, TRIPS0, copy_body, 0)

    for d in range(1, MAX_DEG + 1):
        adj = adjs[d - 1]
        row0 = C0 + (d - 1) * CD
        bufs = [
            (ib0.at[pl.ds(0, G * d)], gb0.at[pl.ds(0, G * d)], gb0, sb0, ob0,
             gsem0, ssem0, osem0),
            (ib1.at[pl.ds(0, G * d)], gb1.at[pl.ds(0, G * d)], gb1, sb1, ob1,
             gsem1, ssem1, osem1),
        ]

        def start(t, p, adj=adj, row0=row0, bufs=bufs, d=d):
            isl, gsl, _, sb, _, gsem, ssem, _ = bufs[p]
            chunk = wid + NW * t

            @pl.when(chunk < NCHUNK)
            def _():
                rbase = row0 + chunk * G
                pltpu.sync_copy(adj.at[pl.ds(chunk * G * d, G * d)], isl)
                pltpu.async_copy(atoms.at[isl], gsl, gsem)
                pltpu.async_copy(atoms.at[pl.ds(rbase, G)], sb, ssem)

        def finish(t, u, p, row0=row0, bufs=bufs, d=d):
            isl, gsl, gb, sb, ob, gsem, ssem, osem = bufs[p]
            chunk = wid + NW * t

            @pl.when(chunk < NCHUNK)
            def _():
                rbase = row0 + chunk * G
                pltpu.make_async_copy(atoms.at[isl], gsl, gsem).wait()
                pltpu.make_async_copy(atoms.at[pl.ds(rbase, G)], sb, ssem).wait()

                @pl.when(u >= 1)
                def _():
                    pltpu.make_async_copy(ob, out.at[pl.ds(row0, G)], osem).wait()

                def row_body(i, c2):
                    for c in range(D // 16):
                        sl = pl.ds(c * 16, 16)
                        v = sb[i, sl]
                        for j in range(d):
                            v = jnp.maximum(v, gb[i * d + j, sl])
                        ob[i, sl] = v
                    return c2

                lax.fori_loop(0, G, row_body, 0)
                pltpu.async_copy(ob, out.at[pl.ds(rbase, G)], osem)

        start(0, 0)
        start(1, 1)

        def pair_body(u, carry):
            t0 = 2 * u
            finish(t0, u, 0)
            start(t0 + 2, 0)
            finish(t0 + 1, u, 1)
            start(t0 + 3, 1)
            return carry

        lax.fori_loop(0, TRIPS // 2, pair_body, 0)

        @pl.when(wid + NW * (TRIPS - 2) < NCHUNK)
        def _(row0=row0, bufs=bufs):
            pltpu.make_async_copy(bufs[0][4], out.at[pl.ds(row0, G)],
                                  bufs[0][7]).wait()

        @pl.when(wid + NW * (TRIPS - 1) < NCHUNK)
        def _(row0=row0, bufs=bufs):
            pltpu.make_async_copy(bufs[1][4], out.at[pl.ds(row0, G)],
                                  bufs[1][7]).wait()


_pool = pl.kernel(
    _body,
    out_type=jax.ShapeDtypeStruct((N, D), jnp.float32),
    mesh=_mesh,
    scratch_types=[
        pltpu.VMEM((G, D), jnp.float32),
        pltpu.VMEM((G, D), jnp.float32),
        pltpu.VMEM((G * MAX_DEG, D), jnp.float32),
        pltpu.VMEM((G * MAX_DEG, D), jnp.float32),
        pltpu.VMEM((G, D), jnp.float32),
        pltpu.VMEM((G, D), jnp.float32),
        pltpu.VMEM((G * MAX_DEG,), jnp.int32),
        pltpu.VMEM((G * MAX_DEG,), jnp.int32),
        pltpu.VMEM((G0, D), jnp.float32),
        pltpu.SemaphoreType.DMA,
        pltpu.SemaphoreType.DMA,
        pltpu.SemaphoreType.DMA,
        pltpu.SemaphoreType.DMA,
        pltpu.SemaphoreType.DMA,
        pltpu.SemaphoreType.DMA,
    ],
)


def kernel(atoms, deg_slice, membership, deg_adj_1, deg_adj_2, deg_adj_3,
           deg_adj_4, deg_adj_5, deg_adj_6, deg_adj_7, deg_adj_8,
           deg_adj_9, deg_adj_10):
    flats = [a.reshape(-1) for a in
             (deg_adj_1, deg_adj_2, deg_adj_3, deg_adj_4, deg_adj_5,
              deg_adj_6, deg_adj_7, deg_adj_8, deg_adj_9, deg_adj_10)]
    return _pool(atoms, *flats)


# speedup vs baseline: 3.6020x; 1.2065x over previous
import jax
import jax.numpy as jnp
from jax import lax
from jax.experimental import pallas as pl
from jax.experimental.pallas import tpu as pltpu
from jax.experimental.pallas import tpu_sc as plsc

N = 100000
D = 128
MAX_DEG = 10
C0 = 10000
CD = 9000

NC = 2
NS = 16
NW = NC * NS

CHUNK_ROWS = {1: 72, 2: 72, 3: 72, 4: 40, 5: 40, 6: 40,
              7: 24, 8: 24, 9: 24, 10: 24}

G0 = 80
NCHUNK0 = C0 // G0
TRIPS0 = (NCHUNK0 + NW - 1) // NW

_mesh = plsc.VectorSubcoreMesh(
    core_axis_name="c", subcore_axis_name="s", num_cores=NC, num_subcores=NS
)


def _body(atoms, adj1, adj2, adj3, adj4, adj5, adj6, adj7, adj8, adj9, adj10,
          out, gsem0, gsem1, ssem0, ssem1, osem0, osem1):
    adjs = [adj1, adj2, adj3, adj4, adj5, adj6, adj7, adj8, adj9, adj10]
    wid = lax.axis_index("s") * NC + lax.axis_index("c")
    gsems = [gsem0, gsem1]
    ssems = [ssem0, ssem1]
    osems = [osem0, osem1]

    def deg0(cb):
        def copy_body(t, carry):
            chunk = wid + NW * t

            @pl.when(chunk < NCHUNK0)
            def _():
                base = chunk * G0
                pltpu.sync_copy(atoms.at[pl.ds(base, G0)], cb)
                pltpu.sync_copy(cb, out.at[pl.ds(base, G0)])

            return carry

        lax.fori_loop(0, TRIPS0, copy_body, 0)

    pl.run_scoped(deg0, pltpu.VMEM((G0, D), jnp.float32))

    for d in range(1, MAX_DEG + 1):
        adj = adjs[d - 1]
        row0 = C0 + (d - 1) * CD
        g = CHUNK_ROWS[d]
        nchunk = CD // g
        trips = (nchunk + NW - 1) // NW

        def degree(ib0, ib1, gb0, gb1, sb0, sb1, ob0, ob1,
                   adj=adj, row0=row0, g=g, nchunk=nchunk, trips=trips, d=d):
            ibs = [ib0, ib1]
            gbs, sbs, obs = [gb0, gb1], [sb0, sb1], [ob0, ob1]

            def start(t, p):
                chunk = wid + NW * t

                @pl.when(chunk < nchunk)
                def _():
                    rbase = row0 + chunk * g
                    pltpu.sync_copy(adj.at[pl.ds(chunk * g * d, g * d)],
                                    ibs[p])
                    pltpu.async_copy(atoms.at[ibs[p]], gbs[p], gsems[p])
                    pltpu.async_copy(atoms.at[pl.ds(rbase, g)], sbs[p], ssems[p])

            def finish(t, u, p):
                chunk = wid + NW * t

                @pl.when(chunk < nchunk)
                def _():
                    rbase = row0 + chunk * g
                    pltpu.make_async_copy(atoms.at[ibs[p]], gbs[p],
                                          gsems[p]).wait()
                    pltpu.make_async_copy(atoms.at[pl.ds(rbase, g)], sbs[p],
                                          ssems[p]).wait()

                    @pl.when(u >= 1)
                    def _():
                        pltpu.make_async_copy(obs[p], out.at[pl.ds(row0, g)],
                                              osems[p]).wait()

                    def row_body(i, c2):
                        for c in range(D // 16):
                            sl = pl.ds(c * 16, 16)
                            v = sbs[p][i, sl]
                            for j in range(d):
                                v = jnp.maximum(v, gbs[p][i * d + j, sl])
                            obs[p][i, sl] = v
                        return c2

                    lax.fori_loop(0, g, row_body, 0)
                    pltpu.async_copy(obs[p], out.at[pl.ds(rbase, g)], osems[p])

            start(0, 0)
            start(1, 1)

            def pair_body(u, carry):
                t0 = 2 * u
                finish(t0, u, 0)
                start(t0 + 2, 0)
                finish(t0 + 1, u, 1)
                start(t0 + 3, 1)
                return carry

            lax.fori_loop(0, trips // 2, pair_body, 0)

            @pl.when(wid + NW * (trips - 2) < nchunk)
            def _():
                pltpu.make_async_copy(obs[0], out.at[pl.ds(row0, g)],
                                      osems[0]).wait()

            @pl.when(wid + NW * (trips - 1) < nchunk)
            def _():
                pltpu.make_async_copy(obs[1], out.at[pl.ds(row0, g)],
                                      osems[1]).wait()

        pl.run_scoped(
            degree,
            pltpu.VMEM((g * d,), jnp.int32),
            pltpu.VMEM((g * d,), jnp.int32),
            pltpu.VMEM((g * d, D), jnp.float32),
            pltpu.VMEM((g * d, D), jnp.float32),
            pltpu.VMEM((g, D), jnp.float32),
            pltpu.VMEM((g, D), jnp.float32),
            pltpu.VMEM((g, D), jnp.float32),
            pltpu.VMEM((g, D), jnp.float32),
        )


_pool = pl.kernel(
    _body,
    out_type=jax.ShapeDtypeStruct((N, D), jnp.float32),
    mesh=_mesh,
    scratch_types=[
        pltpu.SemaphoreType.DMA,
        pltpu.SemaphoreType.DMA,
        pltpu.SemaphoreType.DMA,
        pltpu.SemaphoreType.DMA,
        pltpu.SemaphoreType.DMA,
        pltpu.SemaphoreType.DMA,
    ],
)


def kernel(atoms, deg_slice, membership, deg_adj_1, deg_adj_2, deg_adj_3,
           deg_adj_4, deg_adj_5, deg_adj_6, deg_adj_7, deg_adj_8,
           deg_adj_9, deg_adj_10):
    flats = [a.reshape(-1) for a in
             (deg_adj_1, deg_adj_2, deg_adj_3, deg_adj_4, deg_adj_5,
              deg_adj_6, deg_adj_7, deg_adj_8, deg_adj_9, deg_adj_10)]
    return _pool(atoms, *flats)
